# Initial kernel scaffold; baseline (speedup 1.0000x reference)
import jax
import jax.numpy as jnp
from jax.experimental import pallas as pl


def kernel(connectivity, nodes_free):
    raise NotImplementedError("write your pallas kernel here")


# speedup vs baseline: 4.9029x; 4.9029x over previous
import functools

import jax
import jax.numpy as jnp
from jax import lax
from jax.experimental import pallas as pl
from jax.experimental.pallas import tpu as pltpu
from jax.experimental.pallas import tpu_sc as plsc

_N_NODES = 100000
_DIM = 2
_B_FLAT = 3200000
_NC = 2
_NS = 16
_NW = _NC * _NS
_PER_W = _B_FLAT // _NW
_CHUNK = 4000
_NCH = _PER_W // _CHUNK


def _gather_body(conn_hbm, table_hbm, out_hbm, idx_v, rows_v, table_sp, sem):
    cid = lax.axis_index("c")
    sid = lax.axis_index("s")

    @pl.when(sid == 0)
    def _stage():
        pltpu.sync_copy(table_hbm, table_sp)

    plsc.subcore_barrier()

    wid = sid * _NC + cid
    base = wid * _PER_W

    def chunk(j, carry):
        off = base + j * _CHUNK
        pltpu.sync_copy(conn_hbm.at[pl.ds(off, _CHUNK)], idx_v)
        pltpu.async_copy(table_sp.at[idx_v], rows_v, sem).wait()
        pltpu.sync_copy(rows_v, out_hbm.at[pl.ds(off, _CHUNK)])
        return carry

    lax.fori_loop(0, _NCH, chunk, 0)


@jax.jit
def _grid_gather(conn_flat, table):
    k = pl.kernel(
        _gather_body,
        out_type=jax.ShapeDtypeStruct((_B_FLAT, _DIM), jnp.float32),
        mesh=plsc.VectorSubcoreMesh(core_axis_name="c", subcore_axis_name="s"),
        scratch_types=[
            pltpu.VMEM((_CHUNK,), jnp.int32),
            pltpu.VMEM((_CHUNK, _DIM), jnp.float32),
            pltpu.VMEM_SHARED((_N_NODES, _DIM), jnp.float32),
            pltpu.SemaphoreType.DMA,
        ],
        compiler_params=pltpu.CompilerParams(use_tc_tiling_on_sc=False),
    )
    return k(conn_flat, table)


def kernel(connectivity, nodes_free):
    n_elems, npe = connectivity.shape
    conn_flat = connectivity.reshape(-1)
    out = _grid_gather(conn_flat, nodes_free)
    return out.reshape(n_elems, npe, nodes_free.shape[1])


# speedup vs baseline: 69.1906x; 14.1121x over previous
import functools

import jax
import jax.numpy as jnp
from jax import lax
from jax.experimental import pallas as pl
from jax.experimental.pallas import tpu as pltpu
from jax.experimental.pallas import tpu_sc as plsc

_N_NODES = 100000
_DIM = 2
_N_ELEMS = 800000
_NPE = 4
_NC = 2
_NS = 16
_NW = _NC * _NS
_NQ = _NW // (_NPE * _DIM)
_PER_Q = _N_ELEMS // _NQ
_CHUNK = 8000
_NCH = _PER_Q // _CHUNK
_LANES = 16


def _gather_body(conn_hbm, table_hbm, out_hbm, table_v, idx_v, out_v):
    cid = lax.axis_index("c")
    sid = lax.axis_index("s")
    wid = sid * _NC + cid
    s = wid // (_DIM * _NQ)
    d = (wid // _NQ) % _DIM
    q = wid % _NQ
    e0 = q * _PER_Q

    pltpu.sync_copy(table_hbm.at[pl.ds(d, 1)], table_v)

    def chunk(j, carry):
        off = e0 + j * _CHUNK
        pltpu.sync_copy(conn_hbm.at[pl.ds(s, 1), pl.ds(off, _CHUNK)], idx_v)

        def step(i, carry2):
            ids = idx_v[0, pl.ds(i * _LANES, _LANES)]
            zero16 = jnp.zeros((_LANES,), jnp.int32)
            vals = plsc.load_gather(table_v, [zero16, ids])
            out_v[0, 0, pl.ds(i * _LANES, _LANES)] = vals
            return carry2

        lax.fori_loop(0, _CHUNK // _LANES, step, 0)
        pltpu.sync_copy(out_v,
                        out_hbm.at[pl.ds(s, 1), pl.ds(d, 1), pl.ds(off, _CHUNK)])
        return carry

    lax.fori_loop(0, _NCH, chunk, 0)


@jax.jit
def _grid_gather(conn_t, table_t):
    k = pl.kernel(
        _gather_body,
        out_type=jax.ShapeDtypeStruct((_NPE, _DIM, _N_ELEMS), jnp.float32),
        mesh=plsc.VectorSubcoreMesh(core_axis_name="c", subcore_axis_name="s"),
        scratch_types=[
            pltpu.VMEM((1, _N_NODES), jnp.float32),
            pltpu.VMEM((1, _CHUNK), jnp.int32),
            pltpu.VMEM((1, 1, _CHUNK), jnp.float32),
        ],
        compiler_params=pltpu.CompilerParams(
            use_tc_tiling_on_sc=False, needs_layout_passes=False
        ),
    )
    return k(conn_t, table_t)


def kernel(connectivity, nodes_free):
    conn_t = connectivity.T
    table_t = nodes_free.T
    out_p = _grid_gather(conn_t, table_t)
    return jnp.transpose(out_p, (2, 0, 1))


# speedup vs baseline: 98.4940x; 1.4235x over previous
import functools

import jax
import jax.numpy as jnp
from jax import lax
from jax.experimental import pallas as pl
from jax.experimental.pallas import tpu as pltpu
from jax.experimental.pallas import tpu_sc as plsc

_N_NODES = 100000
_DIM = 2
_N_ELEMS = 800000
_NPE = 4
_NC = 2
_NS = 16
_NW = _NC * _NS
_NQ = _NW // (_NPE * _DIM)
_PER_Q = _N_ELEMS // _NQ
_CHUNK = 6400
_NCH = 32
_LANES = 16
_UNROLL = 10
_STEP = _LANES * _UNROLL


def _gather_body(conn_hbm, table_hbm, out_hbm, table_v, idx2, out2,
                 sem_i0, sem_i1, sem_o0, sem_o1):
    cid = lax.axis_index("c")
    sid = lax.axis_index("s")
    wid = sid * _NC + cid
    s = wid // (_DIM * _NQ)
    d = (wid // _NQ) % _DIM
    q = wid % _NQ
    e0 = q * _PER_Q

    sem_in = (sem_i0, sem_i1)
    sem_out = (sem_o0, sem_o1)

    def off_of(j):
        return e0 + jnp.minimum(j * _CHUNK, _PER_Q - _CHUNK)

    def in_copy(j, p):
        return pltpu.make_async_copy(
            conn_hbm.at[pl.ds(s, 1), pl.ds(off_of(j), _CHUNK)],
            idx2.at[p], sem_in[p])

    def out_copy(j, p):
        return pltpu.make_async_copy(
            out2.at[p],
            out_hbm.at[pl.ds(s, 1), pl.ds(d, 1), pl.ds(off_of(j), _CHUNK)],
            sem_out[p])

    pltpu.sync_copy(table_hbm.at[pl.ds(d, 1)], table_v)

    in_copy(0, 0).start()

    def pair(jj, carry):
        for p in (0, 1):
            j = 2 * jj + p
            @pl.when(j + 1 < _NCH)
            def _pf():
                in_copy(j + 1, 1 - p).start()

            in_copy(j, p).wait()

            @pl.when(j >= 2)
            def _drain():
                out_copy(j - 2, p).wait()

            def step(i, c2):
                b = i * _STEP
                for u in range(_UNROLL):
                    ids = idx2[p, 0, pl.ds(b + u * _LANES, _LANES)]
                    zero16 = jnp.zeros((_LANES,), jnp.int32)
                    vals = plsc.load_gather(table_v, [zero16, ids])
                    out2[p, 0, 0, pl.ds(b + u * _LANES, _LANES)] = vals
                return c2

            lax.fori_loop(0, _CHUNK // _STEP, step, 0)
            out_copy(j, p).start()
        return carry

    lax.fori_loop(0, _NCH // 2, pair, 0)
    out_copy(_NCH - 2, 0).wait()
    out_copy(_NCH - 1, 1).wait()


@jax.jit
def _grid_gather(conn_t, table_t):
    k = pl.kernel(
        _gather_body,
        out_type=jax.ShapeDtypeStruct((_NPE, _DIM, _N_ELEMS), jnp.float32),
        mesh=plsc.VectorSubcoreMesh(core_axis_name="c", subcore_axis_name="s"),
        scratch_types=[
            pltpu.VMEM((1, _N_NODES), jnp.float32),
            pltpu.VMEM((2, 1, _CHUNK), jnp.int32),
            pltpu.VMEM((2, 1, 1, _CHUNK), jnp.float32),
            pltpu.SemaphoreType.DMA,
            pltpu.SemaphoreType.DMA,
            pltpu.SemaphoreType.DMA,
            pltpu.SemaphoreType.DMA,
        ],
        compiler_params=pltpu.CompilerParams(
            use_tc_tiling_on_sc=False, needs_layout_passes=False
        ),
    )
    return k(conn_t, table_t)


def kernel(connectivity, nodes_free):
    conn_t = connectivity.T
    table_t = nodes_free.T
    out_p = _grid_gather(conn_t, table_t)
    return jnp.transpose(out_p, (2, 0, 1))


# speedup vs baseline: 158.7334x; 1.6116x over previous
import functools

import jax
import jax.numpy as jnp
from jax import lax
from jax.experimental import pallas as pl
from jax.experimental.pallas import tpu as pltpu
from jax.experimental.pallas import tpu_sc as plsc

_N_NODES = 100000
_DIM = 2
_N_ELEMS = 800000
_NPE = 4
_LANES = 16
_BLK = 128
_NBLK = _N_ELEMS // _BLK
_NC = 2
_NS = 16
_NW = _NC * _NS
_NQ = _NW // (_NPE * _DIM)
_TB = 48
_NCH = -(-(_NBLK // _NQ + 1) // _TB)


def _gather_body(conn_hbm, table_hbm, out_hbm, table_v, idx2, out2,
                 sem_i0, sem_i1, sem_o0, sem_o1):
    cid = lax.axis_index("c")
    sid = lax.axis_index("s")
    wid = sid * _NC + cid
    s = wid // (_DIM * _NQ)
    d = (wid // _NQ) % _DIM
    q = wid % _NQ
    t_base = 1562 * q + jnp.minimum(q, 2)
    n_blocks = 1563 - (q >= 2).astype(jnp.int32)

    sem_in = (sem_i0, sem_i1)
    sem_out = (sem_o0, sem_o1)

    def t0_of(j):
        return t_base + jnp.minimum(j * _TB, n_blocks - _TB)

    def in_copy(j, p):
        return pltpu.make_async_copy(
            conn_hbm.at[pl.ds(t0_of(j), _TB), pl.ds(s, 1), :],
            idx2.at[p], sem_in[p])

    def out_copy(j, p):
        return pltpu.make_async_copy(
            out2.at[p],
            out_hbm.at[pl.ds(s, 1), pl.ds(t0_of(j), _TB), pl.ds(d, 1), :],
            sem_out[p])

    pltpu.sync_copy(table_hbm.at[pl.ds(d, 1)], table_v)

    in_copy(0, 0).start()

    def pair(jj, carry):
        for p in (0, 1):
            j = 2 * jj + p

            @pl.when(j + 1 < _NCH)
            def _pf():
                in_copy(j + 1, 1 - p).start()

            in_copy(j, p).wait()

            @pl.when(j >= 2)
            def _drain():
                out_copy(j - 2, p).wait()

            def step(tb, c2):
                for u in range(_BLK // _LANES):
                    ids = idx2[p, tb, 0, pl.ds(u * _LANES, _LANES)]
                    zero16 = jnp.zeros((_LANES,), jnp.int32)
                    vals = plsc.load_gather(table_v, [zero16, ids])
                    out2[p, 0, tb, 0, pl.ds(u * _LANES, _LANES)] = vals
                return c2

            lax.fori_loop(0, _TB, step, 0)
            out_copy(j, p).start()
        return carry

    lax.fori_loop(0, _NCH // 2, pair, 0)
    jlast = _NCH - 1
    p = jlast % 2
    in_copy(jlast, p).wait()
    out_copy(jlast - 2, p).wait()
    def step_last(tb, c2):
        for u in range(_BLK // _LANES):
            ids = idx2[p, tb, 0, pl.ds(u * _LANES, _LANES)]
            zero16 = jnp.zeros((_LANES,), jnp.int32)
            vals = plsc.load_gather(table_v, [zero16, ids])
            out2[p, 0, tb, 0, pl.ds(u * _LANES, _LANES)] = vals
        return c2
    lax.fori_loop(0, _TB, step_last, 0)
    out_copy(jlast, p).start()

    out_copy(_NCH - 2, 1 - p).wait()
    out_copy(jlast, p).wait()


@jax.jit
def _grid_gather(conn_b, table_t):
    k = pl.kernel(
        _gather_body,
        out_type=jax.ShapeDtypeStruct((_NPE, _NBLK, _DIM, _BLK), jnp.float32),
        mesh=plsc.VectorSubcoreMesh(core_axis_name="c", subcore_axis_name="s"),
        scratch_types=[
            pltpu.VMEM((1, _N_NODES), jnp.float32),
            pltpu.VMEM((2, _TB, 1, _BLK), jnp.int32),
            pltpu.VMEM((2, 1, _TB, 1, _BLK), jnp.float32),
            pltpu.SemaphoreType.DMA,
            pltpu.SemaphoreType.DMA,
            pltpu.SemaphoreType.DMA,
            pltpu.SemaphoreType.DMA,
        ],
        compiler_params=pltpu.CompilerParams(
            use_tc_tiling_on_sc=False, needs_layout_passes=False
        ),
    )
    return k(conn_b, table_t)


def kernel(connectivity, nodes_free):
    conn_b = connectivity.T.reshape(_NPE, _NBLK, _BLK).transpose(1, 0, 2)
    table_t = nodes_free.T
    out_b = _grid_gather(conn_b, table_t)
    return out_b.transpose(1, 3, 0, 2).reshape(_N_ELEMS, _NPE, _DIM)
